# Initial kernel scaffold; baseline (speedup 1.0000x reference)
import jax
import jax.numpy as jnp
from jax.experimental import pallas as pl


def kernel(h, W_e, b_e, W_g):
    raise NotImplementedError("write your pallas kernel here")


# speedup vs baseline: 1.6944x; 1.6944x over previous
import functools

import jax
import jax.numpy as jnp
from jax.experimental import pallas as pl
from jax.experimental.pallas import tpu as pltpu

HID = 4096
K = 64
TOP_K = 8
ROWS_PER_TILE = 256


def _fused_body(h_ref, wt_ref, be_ref, out_ref):
    hb = h_ref[...]
    p = jnp.dot(hb, wt_ref[...], preferred_element_type=jnp.float32,
                precision=jax.lax.Precision.DEFAULT)
    s = p[:, :K]
    eo = p[:, K:] + be_ref[0, :][None, :]
    m1 = jnp.max(s, axis=-1, keepdims=True)
    t = m1
    cur = s
    for _ in range(TOP_K - 1):
        cur = jnp.where(cur >= t, -jnp.inf, cur)
        t = jnp.max(cur, axis=-1, keepdims=True)
    mask = s >= t
    w = jnp.where(mask, jnp.exp(s - m1), 0.0)
    denom = jnp.sum(w, axis=-1)
    res = jnp.sum(w * eo, axis=-1) / denom
    out_ref[0, 0, :] = res


def kernel(h, W_e, b_e, W_g):
    B, L, hid = h.shape
    rows = B * L
    h2 = h.reshape(rows, hid)
    wt = jnp.concatenate([W_g, W_e], axis=0).T
    be2 = b_e.reshape(1, K)
    n_tiles = rows // ROWS_PER_TILE

    out = pl.pallas_call(
        _fused_body,
        grid=(n_tiles,),
        in_specs=[
            pl.BlockSpec((ROWS_PER_TILE, hid), lambda i: (i, 0)),
            pl.BlockSpec((hid, 2 * K), lambda i: (0, 0)),
            pl.BlockSpec((1, K), lambda i: (0, 0)),
        ],
        out_specs=pl.BlockSpec((1, 1, ROWS_PER_TILE), lambda i: (i, 0, 0)),
        out_shape=jax.ShapeDtypeStruct((n_tiles, 1, ROWS_PER_TILE), jnp.float32),
    )(h2, wt, be2)
    return out.reshape(B, L)


# speedup vs baseline: 2.0872x; 1.2318x over previous
import functools

import jax
import jax.numpy as jnp
from jax.experimental import pallas as pl
from jax.experimental.pallas import tpu as pltpu

HID = 4096
K = 64
TOP_K = 8
ROWS_PER_TILE = 512


def _fused_body(h_ref, wt_ref, be_ref, out_ref):
    hb = h_ref[...]
    p = jnp.dot(hb, wt_ref[...], preferred_element_type=jnp.float32,
                precision=jax.lax.Precision.DEFAULT)
    s = p[:, :K]
    eo = p[:, K:] + be_ref[0, :][None, :]
    m1 = jnp.max(s, axis=-1, keepdims=True)
    t = m1
    cur = s
    for _ in range(TOP_K - 1):
        cur = jnp.where(cur >= t, -jnp.inf, cur)
        t = jnp.max(cur, axis=-1, keepdims=True)
    mask = s >= t
    w = jnp.where(mask, jnp.exp(s - m1), 0.0)
    denom = jnp.sum(w, axis=-1)
    res = jnp.sum(w * eo, axis=-1) / denom
    out_ref[0, 0, :] = res


def kernel(h, W_e, b_e, W_g):
    B, L, hid = h.shape
    rows = B * L
    h2 = h.reshape(rows, hid)
    wt = jnp.concatenate([W_g, W_e], axis=0).T
    be2 = b_e.reshape(1, K)
    n_tiles = rows // ROWS_PER_TILE

    out = pl.pallas_call(
        _fused_body,
        grid=(n_tiles,),
        in_specs=[
            pl.BlockSpec((ROWS_PER_TILE, hid), lambda i: (i, 0)),
            pl.BlockSpec((hid, 2 * K), lambda i: (0, 0)),
            pl.BlockSpec((1, K), lambda i: (0, 0)),
        ],
        out_specs=pl.BlockSpec((1, 1, ROWS_PER_TILE), lambda i: (i, 0, 0)),
        out_shape=jax.ShapeDtypeStruct((n_tiles, 1, ROWS_PER_TILE), jnp.float32),
    )(h2, wt, be2)
    return out.reshape(B, L)


# speedup vs baseline: 2.1190x; 1.0152x over previous
import functools

import jax
import jax.numpy as jnp
from jax.experimental import pallas as pl
from jax.experimental.pallas import tpu as pltpu

HID = 4096
K = 64
TOP_K = 8
ROWS_PER_TILE = 1024


def _fused_body(h_ref, wt_ref, be_ref, out_ref):
    hb = h_ref[...]
    p = jnp.dot(hb, wt_ref[...], preferred_element_type=jnp.float32,
                precision=jax.lax.Precision.DEFAULT)
    s = p[:, :K]
    eo = p[:, K:] + be_ref[0, :][None, :]
    m1 = jnp.max(s, axis=-1, keepdims=True)
    t = m1
    cur = s
    for _ in range(TOP_K - 1):
        cur = jnp.where(cur >= t, -jnp.inf, cur)
        t = jnp.max(cur, axis=-1, keepdims=True)
    mask = s >= t
    w = jnp.where(mask, jnp.exp(s - m1), 0.0)
    denom = jnp.sum(w, axis=-1)
    res = jnp.sum(w * eo, axis=-1) / denom
    out_ref[0, 0, :] = res


def kernel(h, W_e, b_e, W_g):
    B, L, hid = h.shape
    rows = B * L
    h2 = h.reshape(rows, hid)
    wt = jnp.concatenate([W_g, W_e], axis=0).T
    be2 = b_e.reshape(1, K)
    n_tiles = rows // ROWS_PER_TILE

    out = pl.pallas_call(
        _fused_body,
        grid=(n_tiles,),
        in_specs=[
            pl.BlockSpec((ROWS_PER_TILE, hid), lambda i: (i, 0)),
            pl.BlockSpec((hid, 2 * K), lambda i: (0, 0)),
            pl.BlockSpec((1, K), lambda i: (0, 0)),
        ],
        out_specs=pl.BlockSpec((1, 1, ROWS_PER_TILE), lambda i: (i, 0, 0)),
        out_shape=jax.ShapeDtypeStruct((n_tiles, 1, ROWS_PER_TILE), jnp.float32),
    )(h2, wt, be2)
    return out.reshape(B, L)
